# baseline (speedup 1.0000x reference)
import functools

import jax
import jax.numpy as jnp
from jax import lax
from jax.experimental import pallas as pl
from jax.experimental.pallas import tpu as pltpu
from jax.experimental.pallas import tpu_sc as plsc

B = 256
D = 1024
DC = D // 16
C = 16
NEG = -1e30


@functools.cache
def _build(N):
    info = plsc.get_sparse_core_info()
    n_cores, n_sub = info.num_cores, info.num_subcores
    n_workers = n_cores * n_sub
    pairs_per_w = (B // 2) // n_workers

    mesh = plsc.VectorSubcoreMesh(core_axis_name="c", subcore_axis_name="s")

    @functools.partial(
        pl.kernel,
        out_type=(
            jax.ShapeDtypeStruct(((B - 1) * D,), jnp.float32),
            jax.ShapeDtypeStruct(((B - 1) * D,), jnp.float32),
        ),
        mesh=mesh,
        compiler_params=pltpu.CompilerParams(needs_layout_passes=False),
        scratch_types=[
            pltpu.VMEM((C, D), jnp.float32),
            pltpu.VMEM((C, D), jnp.float32),
            pltpu.VMEM((D,), jnp.float32),
            pltpu.VMEM((B + 16,), jnp.int32),
            pltpu.VMEM((D,), jnp.float32),
            pltpu.VMEM((D,), jnp.float32),
            pltpu.SemaphoreType.DMA,
            pltpu.SemaphoreType.DMA,
        ],
    )
    def sc_kernel(x_hbm, sn_hbm, wq_hbm, out_hbm, outseg_hbm,
                  buf0, buf1, wq_v, sn_v, a_v, s_v, sem0, sem1):
        wid = lax.axis_index("s") * n_cores + lax.axis_index("c")
        pltpu.sync_copy(wq_hbm, wq_v)
        pltpu.sync_copy(sn_hbm, sn_v.at[pl.ds(0, B)])
        iota = jnp.arange(16, dtype=jnp.int32)

        def do_segment(seg):
            @pl.when(seg > 0)
            def _():
                seg_len = seg
                r0 = (seg * (seg - 1)) // 2
                base = (r0 // 8) * 8
                n_chunks = (r0 - base + seg_len + C - 1) // C
                n_pairs = (n_chunks + 1) // 2

                def start_of(c):
                    return pl.multiple_of(
                        jnp.minimum(base + c * C, N - C), 8)

                def src_of(c):
                    return x_hbm.at[pl.ds(start_of(c), C), :]

                def zero_body(k, _):
                    sl = pl.ds(k * 16, 16)
                    a_v[sl] = jnp.zeros((16,), jnp.float32)
                    s_v[sl] = jnp.zeros((16,), jnp.float32)
                    return 0
                lax.fori_loop(0, DC, zero_body, 0, unroll=8)

                def process(buf, c, m, svec):
                    start = start_of(c)
                    rows = start + iota

                    zf = jnp.zeros((16,), jnp.float32)

                    def dot_k(k, accs):
                        wqv = wq_v[pl.ds(k * 16, 16)]
                        return tuple(
                            accs[i] + buf[i, pl.ds(k * 16, 16)] * wqv
                            for i in range(16))
                    accs = lax.fori_loop(0, DC, dot_k, (zf,) * 16, unroll=2)

                    valid = ((rows >= jnp.maximum(r0, base + c * C))
                             & (rows < r0 + seg_len))
                    lg = jnp.full((16,), NEG, jnp.float32)
                    for i in range(16):
                        lg = jnp.where(iota == i,
                                       jnp.full((16,), jnp.sum(accs[i]),
                                                jnp.float32), lg)
                    lg = jnp.where(valid, lg,
                                   jnp.full((16,), NEG, jnp.float32))
                    cmax = jnp.max(lg)
                    m_new = jnp.maximum(m, cmax)
                    scalev = jnp.exp(jnp.full((16,), m - m_new, jnp.float32))
                    wg = jnp.exp(lg - m_new)
                    svec_new = svec * scalev + wg
                    vg = jnp.where(valid, jnp.ones((16,), jnp.float32),
                                   jnp.zeros((16,), jnp.float32))
                    wl = [wg[i] for i in range(16)]
                    vl = [vg[i] for i in range(16)]

                    def make_acc_k(masked):
                        def acc_k(k, _):
                            sl = pl.ds(k * 16, 16)
                            aa = [a_v[sl] * scalev, zf, zf, zf]
                            ss = [s_v[sl], zf, zf, zf]
                            for i in range(16):
                                rc = buf[i, pl.ds(k * 16, 16)]
                                aa[i % 4] = aa[i % 4] + rc * wl[i]
                                if masked:
                                    ss[i % 4] = ss[i % 4] + rc * vl[i]
                                else:
                                    ss[i % 4] = ss[i % 4] + rc
                            a_v[sl] = (aa[0] + aa[1]) + (aa[2] + aa[3])
                            s_v[sl] = (ss[0] + ss[1]) + (ss[2] + ss[3])
                            return 0
                        return acc_k

                    full_chunk = ((base + c * C >= r0)
                                  & (base + (c + 1) * C <= r0 + seg_len))

                    @pl.when(full_chunk)
                    def _():
                        lax.fori_loop(0, DC, make_acc_k(False), 0)

                    @pl.when(jnp.logical_not(full_chunk) & (c < n_chunks))
                    def _():
                        lax.fori_loop(0, DC, make_acc_k(True), 0)
                    return m_new, svec_new

                pltpu.async_copy(src_of(0), buf0, sem0)

                def pair_body(t, carry):
                    m, svec = carry
                    c0 = 2 * t
                    c1 = c0 + 1
                    pltpu.make_async_copy(src_of(c0), buf0, sem0).wait()

                    @pl.when(c1 < n_chunks)
                    def _():
                        pltpu.async_copy(src_of(c1), buf1, sem1)
                    m, svec = process(buf0, c0, m, svec)

                    @pl.when(c1 < n_chunks)
                    def _():
                        pltpu.make_async_copy(src_of(c1), buf1, sem1).wait()

                        @pl.when(c1 + 1 < n_chunks)
                        def _():
                            pltpu.async_copy(src_of(c1 + 1), buf0, sem0)
                    m, svec = process(buf1, c1, m, svec)
                    return m, svec

                m, svec = lax.fori_loop(
                    0, n_pairs, pair_body,
                    (jnp.float32(NEG), jnp.zeros((16,), jnp.float32)))

                ones = jnp.ones((16,), jnp.float32)
                inv_a = ones / jnp.full((16,), jnp.sum(svec), jnp.float32)
                cnt = sn_v[pl.ds(seg, 16)][0].astype(jnp.float32)
                inv_s = ones / jnp.maximum(
                    jnp.full((16,), cnt, jnp.float32), 1.0)

                def norm_body(k, _):
                    sl = pl.ds(k * 16, 16)
                    a_v[sl] = a_v[sl] * inv_a
                    s_v[sl] = s_v[sl] * inv_s
                    return 0
                lax.fori_loop(0, DC, norm_body, 0, unroll=8)

                pltpu.sync_copy(a_v, out_hbm.at[pl.ds((seg - 1) * D, D)])
                pltpu.sync_copy(s_v, outseg_hbm.at[pl.ds((seg - 1) * D, D)])

        for q in range(pairs_per_w):
            p = wid * pairs_per_w + q
            do_segment(p)
            do_segment(B - 1 - p)

    return sc_kernel


def kernel(x, segment_num, Wq, bq):
    del bq
    out, out_segment = _build(x.shape[0])(x, segment_num, Wq)
    return out.reshape(B - 1, D), out_segment.reshape(B - 1, D)


# speedup vs baseline: 1.1043x; 1.1043x over previous
import functools

import jax
import jax.numpy as jnp
from jax import lax
from jax.experimental import pallas as pl
from jax.experimental.pallas import tpu as pltpu
from jax.experimental.pallas import tpu_sc as plsc

B = 256
D = 1024
DC = D // 16
C = 16
NEG = -1e30


@functools.cache
def _build(N):
    info = plsc.get_sparse_core_info()
    n_cores, n_sub = info.num_cores, info.num_subcores
    n_workers = n_cores * n_sub
    pairs_per_w = (B // 2) // n_workers

    mesh = plsc.VectorSubcoreMesh(core_axis_name="c", subcore_axis_name="s")

    @functools.partial(
        pl.kernel,
        out_type=(
            jax.ShapeDtypeStruct(((B - 1) * D,), jnp.float32),
            jax.ShapeDtypeStruct(((B - 1) * D,), jnp.float32),
        ),
        mesh=mesh,
        compiler_params=pltpu.CompilerParams(needs_layout_passes=False),
        scratch_types=[
            pltpu.VMEM((C, D), jnp.float32),
            pltpu.VMEM((C, D), jnp.float32),
            pltpu.VMEM((C, D), jnp.float32),
            pltpu.VMEM((D,), jnp.float32),
            pltpu.VMEM((B + 16,), jnp.int32),
            pltpu.VMEM((D,), jnp.float32),
            pltpu.VMEM((D,), jnp.float32),
            pltpu.SemaphoreType.DMA,
            pltpu.SemaphoreType.DMA,
            pltpu.SemaphoreType.DMA,
        ],
    )
    def sc_kernel(x_hbm, sn_hbm, wq_hbm, out_hbm, outseg_hbm,
                  buf0, buf1, buf2, wq_v, sn_v, a_v, s_v, sem0, sem1, sem2):
        wid = lax.axis_index("s") * n_cores + lax.axis_index("c")
        pltpu.sync_copy(wq_hbm, wq_v)
        pltpu.sync_copy(sn_hbm, sn_v.at[pl.ds(0, B)])
        iota = jnp.arange(16, dtype=jnp.int32)

        def src0_of(sg):
            sg_r0 = (sg * (sg - 1)) // 2
            st = pl.multiple_of(
                jnp.minimum((sg_r0 // 8) * 8, N - C), 8)
            return x_hbm.at[pl.ds(st, C), :]

        def do_segment(seg, next_seg):
            pltpu.make_async_copy(src0_of(seg), buf2, sem2).wait()

            @pl.when(seg == 0)
            def _():
                pltpu.async_copy(src0_of(next_seg), buf2, sem2)

            @pl.when(seg > 0)
            def _():
                seg_len = seg
                r0 = (seg * (seg - 1)) // 2
                base = (r0 // 8) * 8
                n_chunks = (r0 - base + seg_len + C - 1) // C

                def start_of(c):
                    return pl.multiple_of(
                        jnp.minimum(base + c * C, N - C), 8)

                def src_of(c):
                    return x_hbm.at[pl.ds(start_of(c), C), :]

                def zero_body(k, _):
                    sl = pl.ds(k * 16, 16)
                    a_v[sl] = jnp.zeros((16,), jnp.float32)
                    s_v[sl] = jnp.zeros((16,), jnp.float32)
                    return 0
                lax.fori_loop(0, DC, zero_body, 0, unroll=8)

                def process(buf, c, m, svec):
                    start = start_of(c)
                    rows = start + iota

                    zf = jnp.zeros((16,), jnp.float32)

                    def dot_k(k, accs):
                        wqv = wq_v[pl.ds(k * 16, 16)]
                        return tuple(
                            accs[i] + buf[i, pl.ds(k * 16, 16)] * wqv
                            for i in range(16))
                    accs = lax.fori_loop(0, DC, dot_k, (zf,) * 16)

                    valid = ((rows >= jnp.maximum(r0, base + c * C))
                             & (rows < r0 + seg_len))
                    lg = jnp.full((16,), NEG, jnp.float32)
                    for i in range(16):
                        lg = jnp.where(iota == i,
                                       jnp.full((16,), jnp.sum(accs[i]),
                                                jnp.float32), lg)
                    lg = jnp.where(valid, lg,
                                   jnp.full((16,), NEG, jnp.float32))
                    cmax = jnp.max(lg)
                    m_new = jnp.maximum(m, cmax)
                    scalev = jnp.exp(jnp.full((16,), m - m_new, jnp.float32))
                    wg = jnp.exp(lg - m_new)
                    svec_new = svec * scalev + wg
                    vg = jnp.where(valid, jnp.ones((16,), jnp.float32),
                                   jnp.zeros((16,), jnp.float32))
                    wl = [wg[i] for i in range(16)]
                    vl = [vg[i] for i in range(16)]

                    def make_acc_k(masked):
                        def acc_k(k, _):
                            sl = pl.ds(k * 16, 16)
                            aa = [a_v[sl] * scalev, zf, zf, zf]
                            ss = [s_v[sl], zf, zf, zf]
                            for i in range(16):
                                rc = buf[i, pl.ds(k * 16, 16)]
                                aa[i % 4] = aa[i % 4] + rc * wl[i]
                                if masked:
                                    ss[i % 4] = ss[i % 4] + rc * vl[i]
                                else:
                                    ss[i % 4] = ss[i % 4] + rc
                            a_v[sl] = (aa[0] + aa[1]) + (aa[2] + aa[3])
                            s_v[sl] = (ss[0] + ss[1]) + (ss[2] + ss[3])
                            return 0
                        return acc_k

                    full_chunk = ((base + c * C >= r0)
                                  & (base + (c + 1) * C <= r0 + seg_len))

                    @pl.when(full_chunk)
                    def _():
                        lax.fori_loop(0, DC, make_acc_k(False), 0)

                    @pl.when(jnp.logical_not(full_chunk) & (c < n_chunks))
                    def _():
                        lax.fori_loop(0, DC, make_acc_k(True), 0)
                    return m_new, svec_new

                @pl.when(n_chunks > 1)
                def _():
                    pltpu.async_copy(src_of(1), buf0, sem0)

                m, svec = process(
                    buf2, 0, jnp.float32(NEG), jnp.zeros((16,), jnp.float32))
                pltpu.async_copy(src0_of(next_seg), buf2, sem2)

                def pair_body(t, carry):
                    m, svec = carry
                    c0 = 2 * t + 1
                    c1 = c0 + 1
                    pltpu.make_async_copy(src_of(c0), buf0, sem0).wait()

                    @pl.when(c1 < n_chunks)
                    def _():
                        pltpu.async_copy(src_of(c1), buf1, sem1)
                    m, svec = process(buf0, c0, m, svec)

                    @pl.when(c1 < n_chunks)
                    def _():
                        pltpu.make_async_copy(src_of(c1), buf1, sem1).wait()

                        @pl.when(c1 + 1 < n_chunks)
                        def _():
                            pltpu.async_copy(src_of(c1 + 1), buf0, sem0)
                    m, svec = process(buf1, c1, m, svec)
                    return m, svec

                m, svec = lax.fori_loop(
                    0, n_chunks // 2, pair_body, (m, svec))

                ones = jnp.ones((16,), jnp.float32)
                inv_a = ones / jnp.full((16,), jnp.sum(svec), jnp.float32)
                cnt = sn_v[pl.ds(seg, 16)][0].astype(jnp.float32)
                inv_s = ones / jnp.maximum(
                    jnp.full((16,), cnt, jnp.float32), 1.0)

                def norm_body(k, _):
                    sl = pl.ds(k * 16, 16)
                    a_v[sl] = a_v[sl] * inv_a
                    s_v[sl] = s_v[sl] * inv_s
                    return 0
                lax.fori_loop(0, DC, norm_body, 0, unroll=8)

                pltpu.sync_copy(a_v, out_hbm.at[pl.ds((seg - 1) * D, D)])
                pltpu.sync_copy(s_v, outseg_hbm.at[pl.ds((seg - 1) * D, D)])

        def seg_at(idx):
            p = wid * pairs_per_w + idx // 2
            return jnp.where(idx % 2 == 0, p, B - 1 - p)

        pltpu.async_copy(src0_of(seg_at(0)), buf2, sem2)

        def seg_body(idx, _):
            do_segment(seg_at(idx), seg_at((idx + 1) % (2 * pairs_per_w)))
            return 0
        lax.fori_loop(0, 2 * pairs_per_w, seg_body, 0)
        pltpu.make_async_copy(src0_of(seg_at(0)), buf2, sem2).wait()

    return sc_kernel


def kernel(x, segment_num, Wq, bq):
    del bq
    out, out_segment = _build(x.shape[0])(x, segment_num, Wq)
    return out.reshape(B - 1, D), out_segment.reshape(B - 1, D)


# speedup vs baseline: 1.1061x; 1.0016x over previous
import functools

import jax
import jax.numpy as jnp
from jax import lax
from jax.experimental import pallas as pl
from jax.experimental.pallas import tpu as pltpu
from jax.experimental.pallas import tpu_sc as plsc

B = 256
D = 1024
DC = D // 16
C = 16
NEG = -1e30


@functools.cache
def _build(N):
    info = plsc.get_sparse_core_info()
    n_cores, n_sub = info.num_cores, info.num_subcores
    n_workers = n_cores * n_sub
    pairs_per_w = (B // 2) // n_workers

    mesh = plsc.VectorSubcoreMesh(core_axis_name="c", subcore_axis_name="s")

    @functools.partial(
        pl.kernel,
        out_type=(
            jax.ShapeDtypeStruct(((B - 1) * D,), jnp.float32),
            jax.ShapeDtypeStruct(((B - 1) * D,), jnp.float32),
        ),
        mesh=mesh,
        compiler_params=pltpu.CompilerParams(needs_layout_passes=False),
        scratch_types=[
            pltpu.VMEM((C, D), jnp.float32),
            pltpu.VMEM((C, D), jnp.float32),
            pltpu.VMEM((C, D), jnp.float32),
            pltpu.VMEM((D,), jnp.float32),
            pltpu.VMEM((B + 16,), jnp.int32),
            pltpu.VMEM((D,), jnp.float32),
            pltpu.VMEM((D,), jnp.float32),
            pltpu.SemaphoreType.DMA,
            pltpu.SemaphoreType.DMA,
            pltpu.SemaphoreType.DMA,
            pltpu.SemaphoreType.DMA,
        ],
    )
    def sc_kernel(x_hbm, sn_hbm, wq_hbm, out_hbm, outseg_hbm,
                  buf0, buf1, buf2, wq_v, sn_v, a_v, s_v,
                  sem0, sem1, sem2, sem3):
        wid = lax.axis_index("s") * n_cores + lax.axis_index("c")
        pltpu.sync_copy(wq_hbm, wq_v)
        pltpu.sync_copy(sn_hbm, sn_v.at[pl.ds(0, B)])
        iota = jnp.arange(16, dtype=jnp.int32)

        def src0_of(sg):
            sg_r0 = (sg * (sg - 1)) // 2
            st = pl.multiple_of(
                jnp.minimum((sg_r0 // 8) * 8, N - C), 8)
            return x_hbm.at[pl.ds(st, C), :]

        def wait_outputs(sg):
            pltpu.make_async_copy(
                a_v, out_hbm.at[pl.ds((sg - 1) * D, D)], sem3).wait()
            pltpu.make_async_copy(
                s_v, outseg_hbm.at[pl.ds((sg - 1) * D, D)], sem3).wait()

        def do_segment(seg, next_seg, prev_seg, idx):
            pltpu.make_async_copy(src0_of(seg), buf2, sem2).wait()

            @pl.when((idx > 0) & (prev_seg > 0))
            def _():
                wait_outputs(prev_seg)

            @pl.when(seg == 0)
            def _():
                pltpu.async_copy(src0_of(next_seg), buf2, sem2)

            @pl.when(seg > 0)
            def _():
                seg_len = seg
                r0 = (seg * (seg - 1)) // 2
                base = (r0 // 8) * 8
                n_chunks = (r0 - base + seg_len + C - 1) // C

                def start_of(c):
                    return pl.multiple_of(
                        jnp.minimum(base + c * C, N - C), 8)

                def src_of(c):
                    return x_hbm.at[pl.ds(start_of(c), C), :]

                def zero_body(k, _):
                    sl = pl.ds(k * 16, 16)
                    a_v[sl] = jnp.zeros((16,), jnp.float32)
                    s_v[sl] = jnp.zeros((16,), jnp.float32)
                    return 0
                lax.fori_loop(0, DC, zero_body, 0, unroll=8)

                def process(buf, c, m, svec):
                    start = start_of(c)
                    rows = start + iota

                    zf = jnp.zeros((16,), jnp.float32)

                    def dot_k(k, accs):
                        wqv = wq_v[pl.ds(k * 16, 16)]
                        return tuple(
                            accs[i] + buf[i, pl.ds(k * 16, 16)] * wqv
                            for i in range(16))
                    accs = lax.fori_loop(0, DC, dot_k, (zf,) * 16)

                    valid = ((rows >= jnp.maximum(r0, base + c * C))
                             & (rows < r0 + seg_len))
                    lg = jnp.full((16,), NEG, jnp.float32)
                    for i in range(16):
                        lg = jnp.where(iota == i,
                                       jnp.full((16,), jnp.sum(accs[i]),
                                                jnp.float32), lg)
                    lg = jnp.where(valid, lg,
                                   jnp.full((16,), NEG, jnp.float32))
                    cmax = jnp.max(lg)
                    m_new = jnp.maximum(m, cmax)
                    scalev = jnp.exp(jnp.full((16,), m - m_new, jnp.float32))
                    wg = jnp.exp(lg - m_new)
                    svec_new = svec * scalev + wg
                    vg = jnp.where(valid, jnp.ones((16,), jnp.float32),
                                   jnp.zeros((16,), jnp.float32))
                    wl = [wg[i] for i in range(16)]
                    vl = [vg[i] for i in range(16)]

                    def make_acc_k(masked):
                        def acc_k(k, _):
                            sl = pl.ds(k * 16, 16)
                            aa = [a_v[sl] * scalev, zf, zf, zf]
                            ss = [s_v[sl], zf, zf, zf]
                            for i in range(16):
                                rc = buf[i, pl.ds(k * 16, 16)]
                                aa[i % 4] = aa[i % 4] + rc * wl[i]
                                if masked:
                                    ss[i % 4] = ss[i % 4] + rc * vl[i]
                                else:
                                    ss[i % 4] = ss[i % 4] + rc
                            a_v[sl] = (aa[0] + aa[1]) + (aa[2] + aa[3])
                            s_v[sl] = (ss[0] + ss[1]) + (ss[2] + ss[3])
                            return 0
                        return acc_k

                    full_chunk = ((base + c * C >= r0)
                                  & (base + (c + 1) * C <= r0 + seg_len))

                    @pl.when(full_chunk)
                    def _():
                        lax.fori_loop(0, DC, make_acc_k(False), 0)

                    @pl.when(jnp.logical_not(full_chunk) & (c < n_chunks))
                    def _():
                        lax.fori_loop(0, DC, make_acc_k(True), 0)
                    return m_new, svec_new

                @pl.when(n_chunks > 1)
                def _():
                    pltpu.async_copy(src_of(1), buf0, sem0)

                m, svec = process(
                    buf2, 0, jnp.float32(NEG), jnp.zeros((16,), jnp.float32))
                pltpu.async_copy(src0_of(next_seg), buf2, sem2)

                def pair_body(t, carry):
                    m, svec = carry
                    c0 = 2 * t + 1
                    c1 = c0 + 1
                    pltpu.make_async_copy(src_of(c0), buf0, sem0).wait()

                    @pl.when(c1 < n_chunks)
                    def _():
                        pltpu.async_copy(src_of(c1), buf1, sem1)
                    m, svec = process(buf0, c0, m, svec)

                    @pl.when(c1 < n_chunks)
                    def _():
                        pltpu.make_async_copy(src_of(c1), buf1, sem1).wait()

                        @pl.when(c1 + 1 < n_chunks)
                        def _():
                            pltpu.async_copy(src_of(c1 + 1), buf0, sem0)
                    m, svec = process(buf1, c1, m, svec)
                    return m, svec

                m, svec = lax.fori_loop(
                    0, n_chunks // 2, pair_body, (m, svec))

                ones = jnp.ones((16,), jnp.float32)
                inv_a = ones / jnp.full((16,), jnp.sum(svec), jnp.float32)
                cnt = sn_v[pl.ds(seg, 16)][0].astype(jnp.float32)
                inv_s = ones / jnp.maximum(
                    jnp.full((16,), cnt, jnp.float32), 1.0)

                def norm_body(k, _):
                    sl = pl.ds(k * 16, 16)
                    a_v[sl] = a_v[sl] * inv_a
                    s_v[sl] = s_v[sl] * inv_s
                    return 0
                lax.fori_loop(0, DC, norm_body, 0, unroll=8)

                pltpu.async_copy(a_v, out_hbm.at[pl.ds((seg - 1) * D, D)],
                                 sem3)
                pltpu.async_copy(s_v, outseg_hbm.at[pl.ds((seg - 1) * D, D)],
                                 sem3)

        def seg_at(idx):
            p = wid * pairs_per_w + idx // 2
            return jnp.where(idx % 2 == 0, p, B - 1 - p)

        pltpu.async_copy(src0_of(seg_at(0)), buf2, sem2)

        def seg_body(idx, _):
            do_segment(seg_at(idx), seg_at((idx + 1) % (2 * pairs_per_w)),
                       seg_at(idx - 1), idx)
            return 0
        lax.fori_loop(0, 2 * pairs_per_w, seg_body, 0)
        pltpu.make_async_copy(src0_of(seg_at(0)), buf2, sem2).wait()
        wait_outputs(seg_at(2 * pairs_per_w - 1))

    return sc_kernel


def kernel(x, segment_num, Wq, bq):
    del bq
    out, out_segment = _build(x.shape[0])(x, segment_num, Wq)
    return out.reshape(B - 1, D), out_segment.reshape(B - 1, D)
